# Initial kernel scaffold; baseline (speedup 1.0000x reference)
import jax
import jax.numpy as jnp
from jax.experimental import pallas as pl


def kernel(q, k, v, offset, Wq, bq, Wk, bk, Wv, bv, W1, b1, W2, b2):
    raise NotImplementedError("write your pallas kernel here")


# speedup vs baseline: 9.5346x; 9.5346x over previous
import functools

import jax
import jax.numpy as jnp
from jax import lax
from jax.experimental import pallas as pl
from jax.experimental.pallas import tpu as pltpu
from jax.experimental.pallas import tpu_sc as plsc

_B = 2
_CLIP = 2
_C = 144
_H = 48
_W = 48
_AREA = _H * _W
_G = 12
_CG = _C // _G
_K2 = 9
_NS = _CLIP * _K2
_BG = _B * _G
_NCH = 9
_CHUNK = _AREA // _NCH
_NBLK = _CHUNK // 16
_KVH = _CLIP * _AREA * _CG
_KVW = 2 * _KVH
_SCALE = float(_CG) ** -0.5

_mesh = plsc.VectorSubcoreMesh(core_axis_name="c", subcore_axis_name="s")


@functools.partial(
    pl.kernel,
    out_type=jax.ShapeDtypeStruct((_BG, _NCH, _CG, _CHUNK), jnp.float32),
    mesh=_mesh,
    scratch_types=[
        pltpu.VMEM((_KVW,), jnp.float32),
        pltpu.VMEM((_NS, 2, _CHUNK), jnp.float32),
        pltpu.VMEM((_CG, _CHUNK), jnp.float32),
        pltpu.VMEM((_CG, _CHUNK), jnp.float32),
        pltpu.VMEM((_NS * 4 * 16,), jnp.float32),
        pltpu.VMEM((_NS * 4 * 16,), jnp.int32),
        pltpu.VMEM((_NS * 16,), jnp.float32),
    ],
    compiler_params=pltpu.CompilerParams(needs_layout_passes=False),
)
def _sc_attn(kv_hbm, off_hbm, q_hbm, out_hbm, kvt, offv, qv, outv, wbuf, ibuf, lbuf):
    wid = lax.axis_index("s") * 2 + lax.axis_index("c")

    @pl.when(wid < _BG)
    def _work():
        bg = wid
        pltpu.sync_copy(kv_hbm.at[bg], kvt)

        def chunk_body(ch, carry0):
            pltpu.sync_copy(off_hbm.at[bg, ch], offv)
            pltpu.sync_copy(q_hbm.at[bg, ch], qv)

            def blk_body(blk, carry1):
                p0 = blk * 16

                def make_s1(clip):
                    base = clip * (_AREA * _CG)

                    def s1(kpos, carry2):
                        s = kpos + clip * _K2
                        sy = offv[s, 0, pl.ds(p0, 16)]
                        sx = offv[s, 1, pl.ds(p0, 16)]
                        ty = sy.astype(jnp.int32)
                        y0 = jnp.where(sy < ty.astype(jnp.float32), ty - 1, ty)
                        tx = sx.astype(jnp.int32)
                        x0 = jnp.where(sx < tx.astype(jnp.float32), tx - 1, tx)
                        gs = []
                        ws = []
                        for t, (dy, dx) in enumerate(((0, 0), (0, 1), (1, 0), (1, 1))):
                            yi = y0 + dy
                            xi = x0 + dx
                            wy = 1.0 - jnp.abs(sy - yi.astype(jnp.float32))
                            wx = 1.0 - jnp.abs(sx - xi.astype(jnp.float32))
                            ok = (yi >= 0) & (yi <= _H - 1) & (xi >= 0) & (xi <= _W - 1)
                            wgt = jnp.where(ok, wy * wx, 0.0)
                            yc = jnp.clip(yi, 0, _H - 1)
                            xc = jnp.clip(xi, 0, _W - 1)
                            gi = base + (yc * _W + xc) * _CG
                            wbuf[pl.ds((s * 4 + t) * 16, 16)] = wgt
                            ibuf[pl.ds((s * 4 + t) * 16, 16)] = gi
                            gs.append(gi)
                            ws.append(wgt)
                        logit = jnp.zeros((16,), jnp.float32)
                        for c in range(_CG):
                            qc = qv[c, pl.ds(p0, 16)]
                            kc = ws[0] * plsc.load_gather(kvt, [gs[0] + c])
                            for t in range(1, 4):
                                kc = kc + ws[t] * plsc.load_gather(kvt, [gs[t] + c])
                            logit = logit + qc * kc
                        lbuf[pl.ds(s * 16, 16)] = logit * _SCALE
                        return carry2

                    return s1

                lax.fori_loop(0, _K2, make_s1(0), 0)
                lax.fori_loop(0, _K2, make_s1(1), 0)

                def smax(s, m):
                    return jnp.maximum(m, lbuf[pl.ds(s * 16, 16)])

                m = lax.fori_loop(1, _NS, smax, lbuf[pl.ds(0, 16)])

                def sexp(s, den):
                    p = jnp.exp(lbuf[pl.ds(s * 16, 16)] - m)
                    lbuf[pl.ds(s * 16, 16)] = p
                    return den + p

                den = lax.fori_loop(0, _NS, sexp, jnp.zeros((16,), jnp.float32))
                rden = 1.0 / den

                def s2(s, acc):
                    a = lbuf[pl.ds(s * 16, 16)] * rden
                    accl = list(acc)
                    for t in range(4):
                        aw = a * wbuf[pl.ds((s * 4 + t) * 16, 16)]
                        gi = ibuf[pl.ds((s * 4 + t) * 16, 16)] + _KVH
                        for c in range(_CG):
                            accl[c] = accl[c] + aw * plsc.load_gather(kvt, [gi + c])
                    return tuple(accl)

                zero = jnp.zeros((16,), jnp.float32)
                acc = lax.fori_loop(0, _NS, s2, (zero,) * _CG)
                for c in range(_CG):
                    outv[c, pl.ds(p0, 16)] = acc[c]
                return carry1

            lax.fori_loop(0, _NBLK, blk_body, 0)
            pltpu.sync_copy(outv, out_hbm.at[bg, ch])
            return carry0

        lax.fori_loop(0, _NCH, chunk_body, 0)


def _proj_body(x_ref, w_ref, b_ref, o_ref):
    o_ref[0] = (
        jnp.dot(x_ref[0], w_ref[0], preferred_element_type=jnp.float32)
        + b_ref[0, 0][None, :]
    )


def _mlp_body(x_ref, w1_ref, b1_ref, w2_ref, b2_ref, o_ref):
    x = x_ref[...]
    h = jnp.dot(x, w1_ref[...], preferred_element_type=jnp.float32) + b1_ref[...]
    h = 0.5 * h * (1.0 + lax.erf(h * (2.0 ** -0.5)))
    o_ref[...] = (
        jnp.dot(h, w2_ref[...], preferred_element_type=jnp.float32)
        + b2_ref[...]
        + x
    )


def _wsel(i):
    return jnp.where(i < 2, 0, jnp.where(i < 6, 1, 2))


def kernel(q, k, v, offset, Wq, bq, Wk, bk, Wv, bv, W1, b1, W2, b2):
    xq = jnp.transpose(q[:, 0], (0, 2, 3, 1)).reshape(_B, _AREA, _C)
    xk = jnp.transpose(k.reshape(_B * _CLIP, _C, _AREA), (0, 2, 1))
    xv = jnp.transpose(v.reshape(_B * _CLIP, _C, _AREA), (0, 2, 1))
    xall = jnp.concatenate([xq, xk, xv], axis=0)
    wall = jnp.stack([Wq.T, Wk.T, Wv.T])
    ball = jnp.stack([bq, bk, bv]).reshape(3, 1, _C)
    proj = pl.pallas_call(
        _proj_body,
        grid=(10,),
        in_specs=[
            pl.BlockSpec((1, _AREA, _C), lambda i: (i, 0, 0)),
            pl.BlockSpec((1, _C, _C), lambda i: (_wsel(i), 0, 0)),
            pl.BlockSpec((1, 1, _C), lambda i: (_wsel(i), 0, 0)),
        ],
        out_specs=pl.BlockSpec((1, _AREA, _C), lambda i: (i, 0, 0)),
        out_shape=jax.ShapeDtypeStruct((10, _AREA, _C), jnp.float32),
    )(xall, wall, ball)

    yq = proj[0:2]
    yk = proj[2:6].reshape(_B, _CLIP, _AREA, _C)
    yv = proj[6:10].reshape(_B, _CLIP, _AREA, _C)
    kk = yk.reshape(_B, _CLIP, _AREA, _G, _CG).transpose(0, 3, 1, 2, 4)
    vv = yv.reshape(_B, _CLIP, _AREA, _G, _CG).transpose(0, 3, 1, 2, 4)
    kvtab = jnp.concatenate(
        [kk.reshape(_BG, 1, _KVH), vv.reshape(_BG, 1, _KVH)], axis=1
    ).reshape(_BG, _KVW)
    qtab = yq.reshape(_B, _NCH, _CHUNK, _G, _CG).transpose(0, 3, 1, 4, 2).reshape(
        _BG, _NCH, _CG, _CHUNK
    )
    pix = jnp.arange(_AREA, dtype=jnp.float32)
    srange = jnp.arange(_NS, dtype=jnp.float32) % _K2
    gy = (pix // _W)[None, :] + (jnp.floor(srange / 3.0) - 1.0)[:, None]
    gx = (pix % _W)[None, :] + (srange % 3.0 - 1.0)[:, None]
    grid = (
        jnp.stack([gy, gx], axis=1)
        .reshape(_NS, 2, _NCH, _CHUNK)
        .transpose(2, 0, 1, 3)
    )
    offtab = (
        offset.reshape(_B, _CLIP, _G, _K2, 2, _NCH, _CHUNK)
        .transpose(0, 2, 5, 1, 3, 4, 6)
        .reshape(_BG, _NCH, _NS, 2, _CHUNK)
        + grid[None]
    )

    aout = _sc_attn(kvtab, offtab, qtab)

    xm = (
        aout.reshape(_B, _G, _NCH, _CG, _CHUNK)
        .transpose(0, 2, 4, 1, 3)
        .reshape(_B * _AREA, _C)
    )
    y = pl.pallas_call(
        _mlp_body,
        grid=(9,),
        in_specs=[
            pl.BlockSpec((512, _C), lambda i: (i, 0)),
            pl.BlockSpec((_C, 2 * _C), lambda i: (0, 0)),
            pl.BlockSpec((1, 2 * _C), lambda i: (0, 0)),
            pl.BlockSpec((2 * _C, _C), lambda i: (0, 0)),
            pl.BlockSpec((1, _C), lambda i: (0, 0)),
        ],
        out_specs=pl.BlockSpec((512, _C), lambda i: (i, 0)),
        out_shape=jax.ShapeDtypeStruct((_B * _AREA, _C), jnp.float32),
    )(xm, W1.T, b1[None, :], W2.T, b2[None, :])

    return y.reshape(_B, _AREA, _C).transpose(0, 2, 1).reshape(_B, 1, _C, _H, _W)


# speedup vs baseline: 15.8573x; 1.6631x over previous
import functools

import jax
import jax.numpy as jnp
from jax import lax
from jax.experimental import pallas as pl
from jax.experimental.pallas import tpu as pltpu
from jax.experimental.pallas import tpu_sc as plsc

_B = 2
_CLIP = 2
_C = 144
_H = 48
_W = 48
_AREA = _H * _W
_G = 12
_CG = _C // _G
_K2 = 9
_NS = _CLIP * _K2
_BG = _B * _G
_NCH = 9
_CHUNK = _AREA // _NCH
_NBLK = _CHUNK // 16
_CLW = _CG * _AREA
_KVW = 4 * _CLW
_SLAB = _C * _AREA
_SCALE = float(_CG) ** -0.5

_mesh = plsc.VectorSubcoreMesh(core_axis_name="c", subcore_axis_name="s")


@functools.partial(
    pl.kernel,
    out_type=jax.ShapeDtypeStruct((_B, _G, _CG, _AREA), jnp.float32),
    mesh=_mesh,
    scratch_types=[
        pltpu.VMEM((_KVW,), jnp.float32),
        pltpu.VMEM((_CLIP, _K2, 2, _CHUNK), jnp.float32),
        pltpu.VMEM((_CG, _CHUNK), jnp.float32),
        pltpu.VMEM((_CG, _CHUNK), jnp.float32),
        pltpu.VMEM((_NS * 4 * 16,), jnp.float32),
        pltpu.VMEM((_NS * 4 * 16,), jnp.int32),
        pltpu.VMEM((_NS * 16,), jnp.float32),
    ],
    compiler_params=pltpu.CompilerParams(needs_layout_passes=False),
)
def _sc_attn(pflat_hbm, proj_hbm, off_hbm, out_hbm, kvt, offv, qv, outv, wbuf, ibuf, lbuf):
    wid = lax.axis_index("s") * 2 + lax.axis_index("c")

    @pl.when(wid < _BG)
    def _work():
        b = wid // _G
        g = wid - b * _G
        c0 = g * _CG
        for clip in range(_CLIP):
            pltpu.sync_copy(
                pflat_hbm.at[pl.ds((2 + b * 2 + clip) * _SLAB + c0 * _AREA, _CLW)],
                kvt.at[pl.ds(clip * _CLW, _CLW)],
            )
            pltpu.sync_copy(
                pflat_hbm.at[pl.ds((6 + b * 2 + clip) * _SLAB + c0 * _AREA, _CLW)],
                kvt.at[pl.ds((2 + clip) * _CLW, _CLW)],
            )

        def chunk_body(ch, carry0):
            for clip in range(_CLIP):
                pltpu.sync_copy(
                    off_hbm.at[b, clip, g, :, :, pl.ds(ch * _CHUNK, _CHUNK)],
                    offv.at[clip],
                )
            pltpu.sync_copy(
                proj_hbm.at[b, g, :, pl.ds(ch * _CHUNK, _CHUNK)], qv
            )

            def blk_body(blk, carry1):
                p0 = blk * 16

                def make_s1(clip):
                    base = clip * _CLW

                    def s1(kpos, carry2):
                        s = kpos + clip * _K2
                        sy = offv[clip, kpos, 0, pl.ds(p0, 16)]
                        sx = offv[clip, kpos, 1, pl.ds(p0, 16)]
                        ty = sy.astype(jnp.int32)
                        y0 = jnp.where(sy < ty.astype(jnp.float32), ty - 1, ty)
                        tx = sx.astype(jnp.int32)
                        x0 = jnp.where(sx < tx.astype(jnp.float32), tx - 1, tx)
                        gs = []
                        ws = []
                        for t, (dy, dx) in enumerate(((0, 0), (0, 1), (1, 0), (1, 1))):
                            yi = y0 + dy
                            xi = x0 + dx
                            wy = 1.0 - jnp.abs(sy - yi.astype(jnp.float32))
                            wx = 1.0 - jnp.abs(sx - xi.astype(jnp.float32))
                            ok = (yi >= 0) & (yi <= _H - 1) & (xi >= 0) & (xi <= _W - 1)
                            wgt = jnp.where(ok, wy * wx, 0.0)
                            yc = jnp.clip(yi, 0, _H - 1)
                            xc = jnp.clip(xi, 0, _W - 1)
                            gi = base + yc * _W + xc
                            wbuf[pl.ds((s * 4 + t) * 16, 16)] = wgt
                            ibuf[pl.ds((s * 4 + t) * 16, 16)] = gi
                            gs.append(gi)
                            ws.append(wgt)
                        logit = jnp.zeros((16,), jnp.float32)
                        for c in range(_CG):
                            qc = qv[c, pl.ds(p0, 16)]
                            kc = ws[0] * plsc.load_gather(kvt, [gs[0] + c * _AREA])
                            for t in range(1, 4):
                                kc = kc + ws[t] * plsc.load_gather(kvt, [gs[t] + c * _AREA])
                            logit = logit + qc * kc
                        lbuf[pl.ds(s * 16, 16)] = logit * _SCALE
                        return carry2

                    return s1

                lax.fori_loop(0, _K2, make_s1(0), 0)
                lax.fori_loop(0, _K2, make_s1(1), 0)

                def smax(s, m):
                    return jnp.maximum(m, lbuf[pl.ds(s * 16, 16)])

                m = lax.fori_loop(1, _NS, smax, lbuf[pl.ds(0, 16)])

                def sexp(s, den):
                    p = jnp.exp(lbuf[pl.ds(s * 16, 16)] - m)
                    lbuf[pl.ds(s * 16, 16)] = p
                    return den + p

                den = lax.fori_loop(0, _NS, sexp, jnp.zeros((16,), jnp.float32))
                rden = 1.0 / den

                def s2(s, acc):
                    a = lbuf[pl.ds(s * 16, 16)] * rden
                    accl = list(acc)
                    for t in range(4):
                        aw = a * wbuf[pl.ds((s * 4 + t) * 16, 16)]
                        gi = ibuf[pl.ds((s * 4 + t) * 16, 16)] + 2 * _CLW
                        for c in range(_CG):
                            accl[c] = accl[c] + aw * plsc.load_gather(
                                kvt, [gi + c * _AREA]
                            )
                    return tuple(accl)

                zero = jnp.zeros((16,), jnp.float32)
                acc = lax.fori_loop(0, _NS, s2, (zero,) * _CG)
                for c in range(_CG):
                    outv[c, pl.ds(p0, 16)] = acc[c]
                return carry1

            lax.fori_loop(0, _NBLK, blk_body, 0)
            pltpu.sync_copy(
                outv, out_hbm.at[b, g, :, pl.ds(ch * _CHUNK, _CHUNK)]
            )
            return carry0

        lax.fori_loop(0, _NCH, chunk_body, 0)


def _proj_body(x_ref, w_ref, b_ref, o_ref):
    o_ref[0] = (
        jnp.dot(w_ref[0], x_ref[0], preferred_element_type=jnp.float32)
        + b_ref[0]
    )


def _mlp_body(x_ref, w1_ref, b1_ref, w2_ref, b2_ref, o_ref):
    x = x_ref[0]
    h = jnp.dot(w1_ref[...], x, preferred_element_type=jnp.float32) + b1_ref[...]
    h = 0.5 * h * (1.0 + lax.erf(h * (2.0 ** -0.5)))
    o_ref[0] = (
        jnp.dot(w2_ref[...], h, preferred_element_type=jnp.float32)
        + b2_ref[...]
        + x
    )


def _wsel(i):
    return jnp.where(i < 2, 0, jnp.where(i < 6, 1, 2))


def kernel(q, k, v, offset, Wq, bq, Wk, bk, Wv, bv, W1, b1, W2, b2):
    xall = jnp.concatenate(
        [
            q.reshape(_B, _C, _AREA),
            k.reshape(_B * _CLIP, _C, _AREA),
            v.reshape(_B * _CLIP, _C, _AREA),
        ],
        axis=0,
    )
    wall = jnp.stack([Wq, Wk, Wv])
    ball = jnp.stack([bq, bk, bv]).reshape(3, _C, 1)
    proj = pl.pallas_call(
        _proj_body,
        grid=(10,),
        in_specs=[
            pl.BlockSpec((1, _C, _AREA), lambda i: (i, 0, 0)),
            pl.BlockSpec((1, _C, _C), lambda i: (_wsel(i), 0, 0)),
            pl.BlockSpec((1, _C, 1), lambda i: (_wsel(i), 0, 0)),
        ],
        out_specs=pl.BlockSpec((1, _C, _AREA), lambda i: (i, 0, 0)),
        out_shape=jax.ShapeDtypeStruct((10, _C, _AREA), jnp.float32),
    )(xall, wall, ball)

    pix = jnp.arange(_AREA, dtype=jnp.float32)
    kr = jnp.arange(_K2, dtype=jnp.float32)[:, None]
    gy = (pix // _W)[None, :] + (jnp.floor(kr / 3.0) - 1.0)
    gx = (pix % _W)[None, :] + (kr % 3.0 - 1.0)
    grid = jnp.stack([gy, gx], axis=1)
    offtab = (
        offset.reshape(_B, _CLIP, _G, _K2, 2, _AREA)
        + grid[None, None, None]
    )

    aout = _sc_attn(
        proj.reshape(10 * _SLAB), proj.reshape(10, _G, _CG, _AREA), offtab
    ).reshape(_B, _C, _AREA)

    y = pl.pallas_call(
        _mlp_body,
        grid=(_B, 3),
        in_specs=[
            pl.BlockSpec((1, _C, _AREA // 3), lambda i, j: (i, 0, j)),
            pl.BlockSpec((2 * _C, _C), lambda i, j: (0, 0)),
            pl.BlockSpec((2 * _C, 1), lambda i, j: (0, 0)),
            pl.BlockSpec((_C, 2 * _C), lambda i, j: (0, 0)),
            pl.BlockSpec((_C, 1), lambda i, j: (0, 0)),
        ],
        out_specs=pl.BlockSpec((1, _C, _AREA // 3), lambda i, j: (i, 0, j)),
        out_shape=jax.ShapeDtypeStruct((_B, _C, _AREA), jnp.float32),
    )(aout, W1, b1[:, None], W2, b2[:, None])

    return y.reshape(_B, 1, _C, _H, _W)


# speedup vs baseline: 18.3630x; 1.1580x over previous
import functools

import jax
import jax.numpy as jnp
from jax import lax
from jax.experimental import pallas as pl
from jax.experimental.pallas import tpu as pltpu
from jax.experimental.pallas import tpu_sc as plsc

_B = 2
_CLIP = 2
_C = 144
_H = 48
_W = 48
_AREA = _H * _W
_G = 12
_CG = _C // _G
_K2 = 9
_NS = _CLIP * _K2
_BG = _B * _G
_NCH = 9
_CHUNK = _AREA // _NCH
_NBLK = _CHUNK // 16
_CLW = _CG * _AREA
_KVW = 4 * _CLW
_SLAB = _C * _AREA
_SCALE = float(_CG) ** -0.5

_mesh = plsc.VectorSubcoreMesh(core_axis_name="c", subcore_axis_name="s")


@functools.partial(
    pl.kernel,
    out_type=jax.ShapeDtypeStruct((_B, _G, _CG, _AREA), jnp.float32),
    mesh=_mesh,
    scratch_types=[
        pltpu.VMEM((_KVW,), jnp.float32),
        pltpu.VMEM((_CLIP, _K2, 2, _CHUNK), jnp.float32),
        pltpu.VMEM((_CG, _CHUNK), jnp.float32),
        pltpu.VMEM((_CG, _CHUNK), jnp.float32),
        pltpu.VMEM((_NS * 4 * 16,), jnp.float32),
        pltpu.VMEM((_NS * 4 * 16,), jnp.int32),
        pltpu.VMEM((_NS * 16,), jnp.float32),
    ],
    compiler_params=pltpu.CompilerParams(needs_layout_passes=False),
)
def _sc_attn(pflat_hbm, proj_hbm, off_hbm, out_hbm, kvt, offv, qv, outv, wbuf, ibuf, lbuf):
    wid = lax.axis_index("s") * 2 + lax.axis_index("c")
    ncg = _BG * _NCH
    start = (wid * ncg) // 32
    end = ((wid + 1) * ncg) // 32

    if True:

        def chunk_body(cg, prev_bg):
            bg = cg // _NCH
            ch = cg - bg * _NCH
            b = bg // _G
            g = bg - b * _G
            c0 = g * _CG

            @pl.when(bg != prev_bg)
            def _load_table():
                for clip in range(_CLIP):
                    pltpu.sync_copy(
                        pflat_hbm.at[
                            pl.ds((2 + b * 2 + clip) * _SLAB + c0 * _AREA, _CLW)
                        ],
                        kvt.at[pl.ds(clip * _CLW, _CLW)],
                    )
                    pltpu.sync_copy(
                        pflat_hbm.at[
                            pl.ds((6 + b * 2 + clip) * _SLAB + c0 * _AREA, _CLW)
                        ],
                        kvt.at[pl.ds((2 + clip) * _CLW, _CLW)],
                    )

            for clip in range(_CLIP):
                pltpu.sync_copy(
                    off_hbm.at[b, clip, g, :, :, pl.ds(ch * _CHUNK, _CHUNK)],
                    offv.at[clip],
                )
            pltpu.sync_copy(
                proj_hbm.at[b, g, :, pl.ds(ch * _CHUNK, _CHUNK)], qv
            )

            def blk_body(blk, carry1):
                p0 = blk * 16
                qs = [qv[c, pl.ds(p0, 16)] for c in range(_CG)]

                def make_s1(clip):
                    base = clip * _CLW

                    def s1(kpos, carry2):
                        s = kpos + clip * _K2
                        sy = offv[clip, kpos, 0, pl.ds(p0, 16)]
                        sx = offv[clip, kpos, 1, pl.ds(p0, 16)]
                        ty = sy.astype(jnp.int32)
                        y0 = jnp.where(sy < ty.astype(jnp.float32), ty - 1, ty)
                        tx = sx.astype(jnp.int32)
                        x0 = jnp.where(sx < tx.astype(jnp.float32), tx - 1, tx)
                        gs = []
                        ws = []
                        for t, (dy, dx) in enumerate(((0, 0), (0, 1), (1, 0), (1, 1))):
                            yi = y0 + dy
                            xi = x0 + dx
                            wy = 1.0 - jnp.abs(sy - yi.astype(jnp.float32))
                            wx = 1.0 - jnp.abs(sx - xi.astype(jnp.float32))
                            ok = (yi >= 0) & (yi <= _H - 1) & (xi >= 0) & (xi <= _W - 1)
                            wgt = jnp.where(ok, wy * wx, 0.0)
                            yc = jnp.clip(yi, 0, _H - 1)
                            xc = jnp.clip(xi, 0, _W - 1)
                            gi = base + yc * _W + xc
                            wbuf[pl.ds((s * 4 + t) * 16, 16)] = wgt
                            ibuf[pl.ds((s * 4 + t) * 16, 16)] = gi
                            gs.append(gi)
                            ws.append(wgt)
                        logit = jnp.zeros((16,), jnp.float32)
                        for c in range(_CG):
                            qc = qs[c]
                            kc = ws[0] * plsc.load_gather(kvt, [gs[0] + c * _AREA])
                            for t in range(1, 4):
                                kc = kc + ws[t] * plsc.load_gather(kvt, [gs[t] + c * _AREA])
                            logit = logit + qc * kc
                        lbuf[pl.ds(s * 16, 16)] = logit * _SCALE
                        return carry2

                    return s1

                lax.fori_loop(0, _K2, make_s1(0), 0)
                lax.fori_loop(0, _K2, make_s1(1), 0)

                def smax(s, m):
                    return jnp.maximum(m, lbuf[pl.ds(s * 16, 16)])

                m = lax.fori_loop(1, _NS, smax, lbuf[pl.ds(0, 16)])

                def sexp(s, den):
                    p = jnp.exp(lbuf[pl.ds(s * 16, 16)] - m)
                    lbuf[pl.ds(s * 16, 16)] = p
                    return den + p

                den = lax.fori_loop(0, _NS, sexp, jnp.zeros((16,), jnp.float32))
                rden = 1.0 / den

                def s2(s, acc):
                    a = lbuf[pl.ds(s * 16, 16)] * rden
                    accl = list(acc)
                    for t in range(4):
                        aw = a * wbuf[pl.ds((s * 4 + t) * 16, 16)]
                        gi = ibuf[pl.ds((s * 4 + t) * 16, 16)] + 2 * _CLW
                        for c in range(_CG):
                            accl[c] = accl[c] + aw * plsc.load_gather(
                                kvt, [gi + c * _AREA]
                            )
                    return tuple(accl)

                zero = jnp.zeros((16,), jnp.float32)
                acc = lax.fori_loop(0, _NS, s2, (zero,) * _CG)
                for c in range(_CG):
                    outv[c, pl.ds(p0, 16)] = acc[c]
                return carry1

            lax.fori_loop(0, _NBLK, blk_body, 0)
            pltpu.sync_copy(
                outv, out_hbm.at[b, g, :, pl.ds(ch * _CHUNK, _CHUNK)]
            )
            return bg

        lax.fori_loop(start, end, chunk_body, jnp.int32(-1))


def _proj_body(x_ref, w_ref, b_ref, o_ref):
    o_ref[0] = (
        jnp.dot(w_ref[0], x_ref[0], preferred_element_type=jnp.float32)
        + b_ref[0]
    )


def _mlp_body(x_ref, w1_ref, b1_ref, w2_ref, b2_ref, o_ref):
    x = x_ref[0]
    h = jnp.dot(w1_ref[...], x, preferred_element_type=jnp.float32) + b1_ref[...]
    h = 0.5 * h * (1.0 + lax.erf(h * (2.0 ** -0.5)))
    o_ref[0] = (
        jnp.dot(w2_ref[...], h, preferred_element_type=jnp.float32)
        + b2_ref[...]
        + x
    )


def _wsel(i):
    return jnp.where(i < 2, 0, jnp.where(i < 6, 1, 2))


def kernel(q, k, v, offset, Wq, bq, Wk, bk, Wv, bv, W1, b1, W2, b2):
    xall = jnp.concatenate(
        [
            q.reshape(_B, _C, _AREA),
            k.reshape(_B * _CLIP, _C, _AREA),
            v.reshape(_B * _CLIP, _C, _AREA),
        ],
        axis=0,
    )
    wall = jnp.stack([Wq, Wk, Wv])
    ball = jnp.stack([bq, bk, bv]).reshape(3, _C, 1)
    proj = pl.pallas_call(
        _proj_body,
        grid=(10,),
        in_specs=[
            pl.BlockSpec((1, _C, _AREA), lambda i: (i, 0, 0)),
            pl.BlockSpec((1, _C, _C), lambda i: (_wsel(i), 0, 0)),
            pl.BlockSpec((1, _C, 1), lambda i: (_wsel(i), 0, 0)),
        ],
        out_specs=pl.BlockSpec((1, _C, _AREA), lambda i: (i, 0, 0)),
        out_shape=jax.ShapeDtypeStruct((10, _C, _AREA), jnp.float32),
    )(xall, wall, ball)

    pix = jnp.arange(_AREA, dtype=jnp.float32)
    kr = jnp.arange(_K2, dtype=jnp.float32)[:, None]
    gy = (pix // _W)[None, :] + (jnp.floor(kr / 3.0) - 1.0)
    gx = (pix % _W)[None, :] + (kr % 3.0 - 1.0)
    grid = jnp.stack([gy, gx], axis=1)
    offtab = (
        offset.reshape(_B, _CLIP, _G, _K2, 2, _AREA)
        + grid[None, None, None]
    )

    aout = _sc_attn(
        proj.reshape(10 * _SLAB), proj.reshape(10, _G, _CG, _AREA), offtab
    ).reshape(_B, _C, _AREA)

    y = pl.pallas_call(
        _mlp_body,
        grid=(_B, 3),
        in_specs=[
            pl.BlockSpec((1, _C, _AREA // 3), lambda i, j: (i, 0, j)),
            pl.BlockSpec((2 * _C, _C), lambda i, j: (0, 0)),
            pl.BlockSpec((2 * _C, 1), lambda i, j: (0, 0)),
            pl.BlockSpec((_C, 2 * _C), lambda i, j: (0, 0)),
            pl.BlockSpec((_C, 1), lambda i, j: (0, 0)),
        ],
        out_specs=pl.BlockSpec((1, _C, _AREA // 3), lambda i, j: (i, 0, j)),
        out_shape=jax.ShapeDtypeStruct((_B, _C, _AREA), jnp.float32),
    )(aout, W1, b1[:, None], W2, b2[:, None])

    return y.reshape(_B, 1, _C, _H, _W)
